# Initial kernel scaffold; baseline (speedup 1.0000x reference)
import jax
import jax.numpy as jnp
from jax.experimental import pallas as pl


def kernel(x, feature1, edge_index, batch, W_lin, b_lin, Wf1a, Wf1b, W_rel, W_root, b_conv, W_lin1, b_lin1, W_lins, b_lins, gn_w, gn_b, gn_a, W_final, b_final):
    raise NotImplementedError("write your pallas kernel here")


# speedup vs baseline: 2.7664x; 2.7664x over previous
import functools

import jax
import jax.numpy as jnp
from jax import lax
from jax.experimental import pallas as pl
from jax.experimental.pallas import tpu as pltpu
from jax.experimental.pallas import tpu_sc as plsc

N = 10000
E = 320000
H = 128
F1 = 12
NG = 64
EPS = 1e-5

NC = 2
NS = 16
CH = 80
EPT = E // (NC * NS)
NCHUNK = EPT // CH
NPAD = 10240
RPT = NPAD // NS
RP = 128
NPIECE = RPT // RP


def _pre_body(x_ref, wlin_ref, blin_ref, wf1b_ref, wf1a_ref, x1_ref, wc_ref):
    xw = lax.dot_general(x_ref[...], wlin_ref[...], (((1,), (1,)), ((), ())),
                         preferred_element_type=jnp.float32) + blin_ref[...]
    x1_ref[...] = xw * jax.nn.sigmoid(xw)
    wc_ref[...] = lax.dot_general(wf1b_ref[...], wf1a_ref[...],
                                  (((1,), (0,)), ((), ())),
                                  preferred_element_type=jnp.float32)


_pre = pl.pallas_call(
    _pre_body,
    out_shape=[
        jax.ShapeDtypeStruct((N, H), jnp.float32),
        jax.ShapeDtypeStruct((H, F1), jnp.float32),
    ],
)


_BE = 8000


def _f1_body(feat_ref, wc_ref, f1_ref):
    f1_ref[...] = lax.dot_general(feat_ref[...], wc_ref[...],
                                  (((1,), (1,)), ((), ())),
                                  preferred_element_type=jnp.float32)


_f1 = pl.pallas_call(
    _f1_body,
    grid=(E // _BE,),
    in_specs=[
        pl.BlockSpec((_BE, F1), lambda i: (i, 0)),
        pl.BlockSpec((H, F1), lambda i: (0, 0)),
    ],
    out_specs=pl.BlockSpec((_BE, H), lambda i: (i, 0)),
    out_shape=jax.ShapeDtypeStruct((E, H), jnp.float32),
)


def _sc_body(x1_hbm, f1_hbm, src_hbm, dst_hbm, out_hbm,
             idx_s, idx_d, f1_v, xg_v, zbuf, acc):
    c = lax.axis_index("c")
    s = lax.axis_index("s")

    def _zrow(r, carry):
        for j in range(H // 16):
            zbuf[r, pl.ds(16 * j, 16)] = jnp.zeros((16,), jnp.float32)
        return carry

    lax.fori_loop(0, RP, _zrow, 0)
    for p in range(NPIECE):
        pltpu.sync_copy(zbuf, acc.at[pl.ds(s * RPT + p * RP, RP)])
    plsc.subcore_barrier()

    e0 = c * (NS * EPT) + s * EPT

    def _chunk(t, carry):
        base = e0 + t * CH
        pltpu.sync_copy(src_hbm.at[pl.ds(base, CH)], idx_s)
        pltpu.sync_copy(dst_hbm.at[pl.ds(base, CH)], idx_d)
        pltpu.sync_copy(f1_hbm.at[pl.ds(base, CH)], f1_v)
        pltpu.sync_copy(x1_hbm.at[idx_s], xg_v)

        def _erow(e, cc):
            for j in range(H // 16):
                sl = pl.ds(16 * j, 16)
                f1_v[e, sl] = f1_v[e, sl] * xg_v[e, sl]
            return cc

        lax.fori_loop(0, CH, _erow, 0)
        pltpu.sync_copy(f1_v, acc.at[idx_d], add=True)
        return carry

    lax.fori_loop(0, NCHUNK, _chunk, 0)
    plsc.subcore_barrier()

    for p in range(NPIECE):
        r0 = s * RPT + p * RP
        pltpu.sync_copy(acc.at[pl.ds(r0, RP)], zbuf)
        pltpu.sync_copy(zbuf, out_hbm.at[pl.ds(c * NPAD + r0, RP)])


@functools.cache
def _make_sc_agg():
    return pl.kernel(
        _sc_body,
        out_type=jax.ShapeDtypeStruct((NC * NPAD, H), jnp.float32),
        mesh=plsc.VectorSubcoreMesh(core_axis_name="c", subcore_axis_name="s"),
        scratch_types=[
            pltpu.VMEM((CH,), jnp.int32),
            pltpu.VMEM((CH,), jnp.int32),
            pltpu.VMEM((CH, H), jnp.float32),
            pltpu.VMEM((CH, H), jnp.float32),
            pltpu.VMEM((RP, H), jnp.float32),
            pltpu.VMEM_SHARED((NPAD, H), jnp.float32),
        ],
    )


def _post_body(agg2_ref, x1_ref, batch_ref, wrel_ref, wroot_ref, bconv_ref,
               wlin1_ref, blin1_ref, wlins_ref, blins_ref, gnw_ref, gnb_ref,
               gna_ref, wfin_ref, bfin_ref, out_ref):
    def dgT(a, w):
        return lax.dot_general(a, w, (((1,), (1,)), ((), ())),
                               preferred_element_type=jnp.float32)

    x1 = x1_ref[...]
    a2 = agg2_ref[...]
    agg = a2[:N] + a2[NPAD:NPAD + N]
    h1 = dgT(agg, wrel_ref[...]) + dgT(x1, wroot_ref[...]) + bconv_ref[...]
    t = dgT(h1, wlin1_ref[...]) + blin1_ref[...]
    h = t * jax.nn.sigmoid(t) + x1
    for i in range(3):
        t = dgT(h, wlins_ref[i]) + blins_ref[i]
        h = t * jax.nn.sigmoid(t) + h

    bt = batch_ref[...]
    gids = lax.broadcasted_iota(jnp.int32, (NG, N), 0)
    oh = (bt == gids).astype(jnp.float32)
    cnt = jnp.maximum(jnp.sum(oh, axis=1, keepdims=True), 1.0)
    mean = lax.dot_general(oh, h, (((1,), (0,)), ((), ())),
                           preferred_element_type=jnp.float32) / cnt
    meann = lax.dot_general(oh, mean, (((0,), (0,)), ((), ())),
                            preferred_element_type=jnp.float32)
    cen = h - gna_ref[...] * meann
    var = lax.dot_general(oh, cen * cen, (((1,), (0,)), ((), ())),
                          preferred_element_type=jnp.float32) / cnt
    varn = lax.dot_general(oh, var, (((0,), (0,)), ((), ())),
                           preferred_element_type=jnp.float32)
    hn = gnw_ref[...] * cen * lax.rsqrt(varn + EPS) + gnb_ref[...]
    out_ref[...] = dgT(hn, wfin_ref[...]) + bfin_ref[...]


_post = pl.pallas_call(
    _post_body,
    out_shape=jax.ShapeDtypeStruct((N, H), jnp.float32),
)


def kernel(x, feature1, edge_index, batch, W_lin, b_lin, Wf1a, Wf1b, W_rel,
           W_root, b_conv, W_lin1, b_lin1, W_lins, b_lins, gn_w, gn_b, gn_a,
           W_final, b_final):
    x1, wc = _pre(x, W_lin, b_lin.reshape(1, H), Wf1b, Wf1a)
    f1 = _f1(feature1, wc)
    agg2 = _make_sc_agg()(x1, f1, edge_index[0], edge_index[1])
    return _post(agg2, x1, batch.reshape(1, N), W_rel, W_root,
                 b_conv.reshape(1, H), W_lin1, b_lin1.reshape(1, H), W_lins,
                 b_lins.reshape(3, 1, H), gn_w.reshape(1, H),
                 gn_b.reshape(1, H), gn_a.reshape(1, H), W_final,
                 b_final.reshape(1, H))


# speedup vs baseline: 4.0147x; 1.4512x over previous
import functools

import jax
import jax.numpy as jnp
from jax import lax
from jax.experimental import pallas as pl
from jax.experimental.pallas import tpu as pltpu
from jax.experimental.pallas import tpu_sc as plsc

N = 10000
E = 320000
H = 128
F1 = 12
NG = 64
EPS = 1e-5

NC = 2
NS = 16
CH = 80
EPT = E // (NC * NS)
NCHUNK = EPT // CH
NPAD = 10240
RPT = NPAD // NS
RP = CH
NPIECE = RPT // RP


def _pre_body(x_ref, wlin_ref, blin_ref, wf1b_ref, wf1a_ref, x1_ref, wc_ref):
    xw = lax.dot_general(x_ref[...], wlin_ref[...], (((1,), (1,)), ((), ())),
                         preferred_element_type=jnp.float32) + blin_ref[...]
    x1_ref[...] = xw * jax.nn.sigmoid(xw)
    wc_ref[...] = lax.dot_general(wf1b_ref[...], wf1a_ref[...],
                                  (((1,), (0,)), ((), ())),
                                  preferred_element_type=jnp.float32)


_pre = pl.pallas_call(
    _pre_body,
    out_shape=[
        jax.ShapeDtypeStruct((N, H), jnp.float32),
        jax.ShapeDtypeStruct((H, F1), jnp.float32),
    ],
)


_BE = 8000


def _f1_body(feat_ref, wc_ref, f1_ref):
    f1_ref[...] = lax.dot_general(feat_ref[...], wc_ref[...],
                                  (((1,), (1,)), ((), ())),
                                  preferred_element_type=jnp.float32)


_f1 = pl.pallas_call(
    _f1_body,
    grid=(E // _BE,),
    in_specs=[
        pl.BlockSpec((_BE, F1), lambda i: (i, 0)),
        pl.BlockSpec((H, F1), lambda i: (0, 0)),
    ],
    out_specs=pl.BlockSpec((_BE, H), lambda i: (i, 0)),
    out_shape=jax.ShapeDtypeStruct((E, H), jnp.float32),
)


NPAIR = NCHUNK // 2


def _sc_body(x1_hbm, f1_hbm, src_hbm, dst_hbm, out_hbm,
             isrc0, idst0, f1b0, xgb0, isrc1, idst1, f1b1, xgb1, acc,
             semA0, semG0, semS0, semA1, semG1, semS1):
    c = lax.axis_index("c")
    s = lax.axis_index("s")
    isrc = (isrc0, isrc1)
    idst = (idst0, idst1)
    f1b = (f1b0, f1b1)
    xgb = (xgb0, xgb1)
    semA = (semA0, semA1)
    semG = (semG0, semG1)
    semS = (semS0, semS1)

    @plsc.parallel_loop(0, RP, 1, unroll=4)
    def _zrow(r):
        for j in range(H // 16):
            f1b0[r, pl.ds(16 * j, 16)] = jnp.zeros((16,), jnp.float32)

    for p in range(NPIECE):
        pltpu.sync_copy(f1b0, acc.at[pl.ds(s * RPT + p * RP, RP)])
    plsc.subcore_barrier()

    e0 = c * (NS * EPT) + s * EPT

    def startA(t, b):
        base = e0 + t * CH
        pltpu.async_copy(src_hbm.at[pl.ds(base, CH)], isrc[b], semA[b])
        pltpu.async_copy(dst_hbm.at[pl.ds(base, CH)], idst[b], semA[b])
        pltpu.async_copy(f1_hbm.at[pl.ds(base, CH)], f1b[b], semA[b])

    def waitA(t, b):
        base = e0 + t * CH
        pltpu.make_async_copy(src_hbm.at[pl.ds(base, CH)], isrc[b], semA[b]).wait()
        pltpu.make_async_copy(dst_hbm.at[pl.ds(base, CH)], idst[b], semA[b]).wait()
        pltpu.make_async_copy(f1_hbm.at[pl.ds(base, CH)], f1b[b], semA[b]).wait()

    def startG(b):
        pltpu.async_copy(x1_hbm.at[isrc[b]], xgb[b], semG[b])

    def waitG(b):
        pltpu.make_async_copy(x1_hbm.at[isrc[b]], xgb[b], semG[b]).wait()

    def startS(b):
        pltpu.async_copy(f1b[b], acc.at[idst[b]], semS[b], add=True)

    def waitS(b):
        pltpu.make_async_copy(f1b[b], acc.at[idst[b]], semS[b]).wait()

    def compute(b):
        fb = f1b[b]
        xb = xgb[b]

        @plsc.parallel_loop(0, CH, 1, unroll=2)
        def _erow(e):
            for j in range(H // 16):
                sl = pl.ds(16 * j, 16)
                fb[e, sl] = fb[e, sl] * xb[e, sl]

    startA(0, 0)
    startA(1, 1)
    waitA(0, 0)
    startG(0)

    def _pair(g, carry):
        c0 = 2 * g
        waitA(c0 + 1, 1)
        startG(1)
        waitG(0)
        compute(0)
        startS(0)
        waitG(1)
        compute(1)
        startS(1)
        waitS(0)
        startA(c0 + 2, 0)
        waitA(c0 + 2, 0)
        startG(0)
        waitS(1)

        @pl.when(g < NPAIR - 1)
        def _():
            startA(c0 + 3, 1)

        return carry

    lax.fori_loop(0, NPAIR, _pair, 0)
    waitG(0)
    compute(0)
    startS(0)
    waitS(0)
    plsc.subcore_barrier()

    for p in range(NPIECE):
        r0 = s * RPT + p * RP
        pltpu.sync_copy(acc.at[pl.ds(r0, RP)], f1b0)
        pltpu.sync_copy(f1b0, out_hbm.at[pl.ds(c * NPAD + r0, RP)])


@functools.cache
def _make_sc_agg():
    return pl.kernel(
        _sc_body,
        out_type=jax.ShapeDtypeStruct((NC * NPAD, H), jnp.float32),
        mesh=plsc.VectorSubcoreMesh(core_axis_name="c", subcore_axis_name="s"),
        scratch_types=[
            pltpu.VMEM((CH,), jnp.int32),
            pltpu.VMEM((CH,), jnp.int32),
            pltpu.VMEM((CH, H), jnp.float32),
            pltpu.VMEM((CH, H), jnp.float32),
            pltpu.VMEM((CH,), jnp.int32),
            pltpu.VMEM((CH,), jnp.int32),
            pltpu.VMEM((CH, H), jnp.float32),
            pltpu.VMEM((CH, H), jnp.float32),
            pltpu.VMEM_SHARED((NPAD, H), jnp.float32),
            pltpu.SemaphoreType.DMA,
            pltpu.SemaphoreType.DMA,
            pltpu.SemaphoreType.DMA,
            pltpu.SemaphoreType.DMA,
            pltpu.SemaphoreType.DMA,
            pltpu.SemaphoreType.DMA,
        ],
    )


def _post_body(agg2_ref, x1_ref, batch_ref, wrel_ref, wroot_ref, bconv_ref,
               wlin1_ref, blin1_ref, wlins_ref, blins_ref, gnw_ref, gnb_ref,
               gna_ref, wfin_ref, bfin_ref, out_ref):
    def dgT(a, w):
        return lax.dot_general(a, w, (((1,), (1,)), ((), ())),
                               preferred_element_type=jnp.float32)

    x1 = x1_ref[...]
    a2 = agg2_ref[...]
    agg = a2[:N] + a2[NPAD:NPAD + N]
    h1 = dgT(agg, wrel_ref[...]) + dgT(x1, wroot_ref[...]) + bconv_ref[...]
    t = dgT(h1, wlin1_ref[...]) + blin1_ref[...]
    h = t * jax.nn.sigmoid(t) + x1
    for i in range(3):
        t = dgT(h, wlins_ref[i]) + blins_ref[i]
        h = t * jax.nn.sigmoid(t) + h

    bt = batch_ref[...]
    gids = lax.broadcasted_iota(jnp.int32, (NG, N), 0)
    oh = (bt == gids).astype(jnp.float32)
    cnt = jnp.maximum(jnp.sum(oh, axis=1, keepdims=True), 1.0)
    mean = lax.dot_general(oh, h, (((1,), (0,)), ((), ())),
                           preferred_element_type=jnp.float32) / cnt
    meann = lax.dot_general(oh, mean, (((0,), (0,)), ((), ())),
                            preferred_element_type=jnp.float32)
    cen = h - gna_ref[...] * meann
    var = lax.dot_general(oh, cen * cen, (((1,), (0,)), ((), ())),
                          preferred_element_type=jnp.float32) / cnt
    varn = lax.dot_general(oh, var, (((0,), (0,)), ((), ())),
                           preferred_element_type=jnp.float32)
    hn = gnw_ref[...] * cen * lax.rsqrt(varn + EPS) + gnb_ref[...]
    out_ref[...] = dgT(hn, wfin_ref[...]) + bfin_ref[...]


_post = pl.pallas_call(
    _post_body,
    out_shape=jax.ShapeDtypeStruct((N, H), jnp.float32),
)


def kernel(x, feature1, edge_index, batch, W_lin, b_lin, Wf1a, Wf1b, W_rel,
           W_root, b_conv, W_lin1, b_lin1, W_lins, b_lins, gn_w, gn_b, gn_a,
           W_final, b_final):
    x1, wc = _pre(x, W_lin, b_lin.reshape(1, H), Wf1b, Wf1a)
    f1 = _f1(feature1, wc)
    agg2 = _make_sc_agg()(x1, f1, edge_index[0], edge_index[1])
    return _post(agg2, x1, batch.reshape(1, N), W_rel, W_root,
                 b_conv.reshape(1, H), W_lin1, b_lin1.reshape(1, H), W_lins,
                 b_lins.reshape(3, 1, H), gn_w.reshape(1, H),
                 gn_b.reshape(1, H), gn_a.reshape(1, H), W_final,
                 b_final.reshape(1, H))


# speedup vs baseline: 5.7702x; 1.4373x over previous
import functools

import jax
import jax.numpy as jnp
from jax import lax
from jax.experimental import pallas as pl
from jax.experimental.pallas import tpu as pltpu
from jax.experimental.pallas import tpu_sc as plsc

N = 10000
E = 320000
H = 128
F1 = 12
NG = 64
EPS = 1e-5

NC = 2
NS = 16
CH = 80
EPT = E // (NC * NS)
NCHUNK = EPT // CH
NPAD = 10240
RPT = NPAD // NS
RP = CH
NPIECE = RPT // RP


def _pre_body(x_ref, wlin_ref, blin_ref, wf1b_ref, wf1a_ref, x1_ref, wc_ref):
    xw = lax.dot_general(x_ref[...], wlin_ref[...], (((1,), (1,)), ((), ())),
                         preferred_element_type=jnp.float32) + blin_ref[...]
    x1_ref[...] = xw * jax.nn.sigmoid(xw)
    wc_ref[...] = lax.dot_general(wf1b_ref[...], wf1a_ref[...],
                                  (((1,), (0,)), ((), ())),
                                  preferred_element_type=jnp.float32)


_pre = pl.pallas_call(
    _pre_body,
    out_shape=[
        jax.ShapeDtypeStruct((N, H), jnp.float32),
        jax.ShapeDtypeStruct((H, F1), jnp.float32),
    ],
)


_BE = 12800


def _bf16_bits(x):
    u16 = jnp.uint32(16)
    u = lax.bitcast_convert_type(x, jnp.uint32)
    rnd = (lax.shift_right_logical(u, u16) & jnp.uint32(1)) + jnp.uint32(0x7FFF)
    return lax.shift_right_logical(u + rnd, u16)


def _f1_body(feat_ref, wc_ref, f1_ref):
    f1_ref[...] = lax.dot_general(feat_ref[...], wc_ref[...],
                                  (((0,), (1,)), ((), ())),
                                  preferred_element_type=jnp.float32)


_f1 = pl.pallas_call(
    _f1_body,
    grid=(E // _BE,),
    in_specs=[
        pl.BlockSpec((F1, _BE), lambda i: (0, i)),
        pl.BlockSpec((H, F1), lambda i: (0, 0)),
    ],
    out_specs=pl.BlockSpec((_BE, H), lambda i: (i, 0)),
    out_shape=jax.ShapeDtypeStruct((E, H), jnp.float32),
)


NPAIR = NCHUNK // 2


def _sc_body(x1_hbm, f1_hbm, src_hbm, dst_hbm, out_hbm,
             isrc0, idst0, f1b0, xgb0, isrc1, idst1, f1b1, xgb1, acc,
             semA0, semG0, semS0, semA1, semG1, semS1):
    c = lax.axis_index("c")
    s = lax.axis_index("s")
    isrc = (isrc0, isrc1)
    idst = (idst0, idst1)
    f1b = (f1b0, f1b1)
    xgb = (xgb0, xgb1)
    semA = (semA0, semA1)
    semG = (semG0, semG1)
    semS = (semS0, semS1)

    @plsc.parallel_loop(0, RP, 1, unroll=4)
    def _zrow(r):
        for j in range(H // 16):
            xgb0[r, pl.ds(16 * j, 16)] = jnp.zeros((16,), jnp.float32)

    for p in range(NPIECE):
        pltpu.sync_copy(xgb0, acc.at[pl.ds(s * RPT + p * RP, RP)])
    plsc.subcore_barrier()

    e0 = c * (NS * EPT) + s * EPT

    def startA(t, b):
        base = e0 + t * CH
        pltpu.async_copy(src_hbm.at[pl.ds(base, CH)], isrc[b], semA[b])
        pltpu.async_copy(dst_hbm.at[pl.ds(base, CH)], idst[b], semA[b])
        pltpu.async_copy(f1_hbm.at[pl.ds(base, CH)], f1b[b], semA[b])

    def waitA(t, b):
        base = e0 + t * CH
        pltpu.make_async_copy(src_hbm.at[pl.ds(base, CH)], isrc[b], semA[b]).wait()
        pltpu.make_async_copy(dst_hbm.at[pl.ds(base, CH)], idst[b], semA[b]).wait()
        pltpu.make_async_copy(f1_hbm.at[pl.ds(base, CH)], f1b[b], semA[b]).wait()

    def startG(b):
        pltpu.async_copy(x1_hbm.at[isrc[b]], xgb[b], semG[b])

    def waitG(b):
        pltpu.make_async_copy(x1_hbm.at[isrc[b]], xgb[b], semG[b]).wait()

    def startS(b):
        pltpu.async_copy(xgb[b], acc.at[idst[b]], semS[b], add=True)

    def waitS(b):
        pltpu.make_async_copy(xgb[b], acc.at[idst[b]], semS[b]).wait()

    def compute(b):
        fb = f1b[b]
        xb = xgb[b]

        @plsc.parallel_loop(0, CH, 1, unroll=4)
        def _erow(e):
            for j in range(H // 16):
                sl = pl.ds(16 * j, 16)
                xb[e, sl] = xb[e, sl] * fb[e, sl]

    startA(0, 0)
    startA(1, 1)
    waitA(0, 0)
    startG(0)

    def _pair(g, carry):
        c0 = 2 * g
        waitA(c0 + 1, 1)
        startG(1)
        waitG(0)
        compute(0)
        startS(0)
        waitS(0)
        startA(c0 + 2, 0)
        waitG(1)
        compute(1)
        startS(1)
        waitA(c0 + 2, 0)
        startG(0)
        waitS(1)

        @pl.when(g < NPAIR - 1)
        def _():
            startA(c0 + 3, 1)

        return carry

    lax.fori_loop(0, NPAIR, _pair, 0)
    waitG(0)
    compute(0)
    startS(0)
    waitS(0)
    plsc.subcore_barrier()

    for p in range(NPIECE):
        r0 = s * RPT + p * RP
        pltpu.sync_copy(acc.at[pl.ds(r0, RP)], xgb0)
        pltpu.sync_copy(xgb0, out_hbm.at[pl.ds(c * NPAD + r0, RP)])


@functools.cache
def _make_sc_agg():
    return pl.kernel(
        _sc_body,
        out_type=jax.ShapeDtypeStruct((NC * NPAD, H), jnp.float32),
        mesh=plsc.VectorSubcoreMesh(core_axis_name="c", subcore_axis_name="s"),
        scratch_types=[
            pltpu.VMEM((CH,), jnp.int32),
            pltpu.VMEM((CH,), jnp.int32),
            pltpu.VMEM((CH, H), jnp.float32),
            pltpu.VMEM((CH, H), jnp.float32),
            pltpu.VMEM((CH,), jnp.int32),
            pltpu.VMEM((CH,), jnp.int32),
            pltpu.VMEM((CH, H), jnp.float32),
            pltpu.VMEM((CH, H), jnp.float32),
            pltpu.VMEM_SHARED((NPAD, H), jnp.float32),
            pltpu.SemaphoreType.DMA,
            pltpu.SemaphoreType.DMA,
            pltpu.SemaphoreType.DMA,
            pltpu.SemaphoreType.DMA,
            pltpu.SemaphoreType.DMA,
            pltpu.SemaphoreType.DMA,
        ],
    )


def _post_body(agg2_ref, x1_ref, batch_ref, wrel_ref, wroot_ref, bconv_ref,
               wlin1_ref, blin1_ref, wlins_ref, blins_ref, gnw_ref, gnb_ref,
               gna_ref, wfin_ref, bfin_ref, out_ref):
    def dgT(a, w):
        return lax.dot_general(a, w, (((1,), (1,)), ((), ())),
                               preferred_element_type=jnp.float32)

    x1 = x1_ref[...]
    a2 = agg2_ref[...]
    agg = a2[:N] + a2[NPAD:NPAD + N]
    h1 = dgT(agg, wrel_ref[...]) + dgT(x1, wroot_ref[...]) + bconv_ref[...]
    t = dgT(h1, wlin1_ref[...]) + blin1_ref[...]
    h = t * jax.nn.sigmoid(t) + x1
    for i in range(3):
        t = dgT(h, wlins_ref[i]) + blins_ref[i]
        h = t * jax.nn.sigmoid(t) + h

    bt = batch_ref[...]
    gids = lax.broadcasted_iota(jnp.int32, (NG, N), 0)
    oh = (bt == gids).astype(jnp.float32)
    cnt = jnp.maximum(jnp.sum(oh, axis=1, keepdims=True), 1.0)
    mean = lax.dot_general(oh, h, (((1,), (0,)), ((), ())),
                           preferred_element_type=jnp.float32) / cnt
    meann = lax.dot_general(oh, mean, (((0,), (0,)), ((), ())),
                            preferred_element_type=jnp.float32)
    cen = h - gna_ref[...] * meann
    var = lax.dot_general(oh, cen * cen, (((1,), (0,)), ((), ())),
                          preferred_element_type=jnp.float32) / cnt
    varn = lax.dot_general(oh, var, (((0,), (0,)), ((), ())),
                           preferred_element_type=jnp.float32)
    hn = gnw_ref[...] * cen * lax.rsqrt(varn + EPS) + gnb_ref[...]
    out_ref[...] = dgT(hn, wfin_ref[...]) + bfin_ref[...]


_post = pl.pallas_call(
    _post_body,
    out_shape=jax.ShapeDtypeStruct((N, H), jnp.float32),
)


def kernel(x, feature1, edge_index, batch, W_lin, b_lin, Wf1a, Wf1b, W_rel,
           W_root, b_conv, W_lin1, b_lin1, W_lins, b_lins, gn_w, gn_b, gn_a,
           W_final, b_final):
    x1, wc = _pre(x, W_lin, b_lin.reshape(1, H), Wf1b, Wf1a)
    f1 = _f1(feature1.T, wc)
    agg2 = _make_sc_agg()(x1, f1, edge_index[0], edge_index[1])
    return _post(agg2, x1, batch.reshape(1, N), W_rel, W_root,
                 b_conv.reshape(1, H), W_lin1, b_lin1.reshape(1, H), W_lins,
                 b_lins.reshape(3, 1, H), gn_w.reshape(1, H),
                 gn_b.reshape(1, H), gn_a.reshape(1, H), W_final,
                 b_final.reshape(1, H))
